# baseline (speedup 1.0000x reference)
import functools
import jax
import jax.numpy as jnp
from jax import lax
from jax.experimental import pallas as pl
from jax.experimental.pallas import tpu as pltpu
from jax.experimental.pallas import tpu_sc as plsc

_NC = 2
_NS = 16
_NW = _NC * _NS
_CHUNK = 64


def _sc_body(batch, num_types, dim, emb_hbm, out_hbm, buf, sem):
    c = lax.axis_index("c")
    s = lax.axis_index("s")
    wid = s * _NC + c
    rows_per_tile = batch // _NW
    stage = [pltpu.async_copy(emb_hbm, buf.at[r], sem) for r in range(_CHUNK)]
    for cp in stage:
        cp.wait()
    base = wid * rows_per_tile
    outs = [
        pltpu.async_copy(
            buf, out_hbm.at[pl.ds(base + i * _CHUNK, _CHUNK)], sem
        )
        for i in range(rows_per_tile // _CHUNK)
    ]
    for cp in outs:
        cp.wait()


def kernel(action_mask, embedding):
    batch = action_mask.shape[0]
    num_types, dim = embedding.shape
    mesh = plsc.VectorSubcoreMesh(core_axis_name="c", subcore_axis_name="s")
    body = functools.partial(_sc_body, batch, num_types, dim)
    run = pl.kernel(
        body,
        out_type=jax.ShapeDtypeStruct((batch, num_types, dim), embedding.dtype),
        mesh=mesh,
        scratch_types=[
            pltpu.VMEM((_CHUNK, num_types, dim), embedding.dtype),
            pltpu.SemaphoreType.DMA,
        ],
    )
    return run(embedding)


# speedup vs baseline: 1.6257x; 1.6257x over previous
import functools
import jax
import jax.numpy as jnp
from jax import lax
from jax.experimental import pallas as pl
from jax.experimental.pallas import tpu as pltpu
from jax.experimental.pallas import tpu_sc as plsc

_NC = 2
_NS = 16
_NW = _NC * _NS
_CHUNK = 32
_LANES = 16


def _sc_body(batch, num_types, dim, emb_hbm, out_hbm, buf, sem):
    c = lax.axis_index("c")
    s = lax.axis_index("s")
    wid = s * _NC + c
    rows_per_tile = batch // _NW
    pltpu.async_copy(emb_hbm, buf.at[0], sem).wait()
    nvec = dim // _LANES
    regs = [
        buf[0, t, pl.ds(l * _LANES, _LANES)]
        for t in range(num_types)
        for l in range(nvec)
    ]

    def _rep(r, carry):
        for t in range(num_types):
            for l in range(nvec):
                buf[r, t, pl.ds(l * _LANES, _LANES)] = regs[t * nvec + l]
        return carry

    lax.fori_loop(1, _CHUNK, _rep, 0)

    base = wid * rows_per_tile
    outs = [
        pltpu.async_copy(
            buf, out_hbm.at[pl.ds(base + i * _CHUNK, _CHUNK)], sem
        )
        for i in range(rows_per_tile // _CHUNK)
    ]
    for cp in outs:
        cp.wait()


def kernel(action_mask, embedding):
    batch = action_mask.shape[0]
    num_types, dim = embedding.shape
    mesh = plsc.VectorSubcoreMesh(core_axis_name="c", subcore_axis_name="s")
    body = functools.partial(_sc_body, batch, num_types, dim)
    run = pl.kernel(
        body,
        out_type=jax.ShapeDtypeStruct((batch, num_types, dim), embedding.dtype),
        mesh=mesh,
        scratch_types=[
            pltpu.VMEM((_CHUNK, num_types, dim), embedding.dtype),
            pltpu.SemaphoreType.DMA,
        ],
        compiler_params=pltpu.CompilerParams(use_tc_tiling_on_sc=True),
    )
    return run(embedding)


# speedup vs baseline: 3.6602x; 2.2515x over previous
import functools
import jax
import jax.numpy as jnp
from jax import lax
from jax.experimental import pallas as pl
from jax.experimental.pallas import tpu as pltpu
from jax.experimental.pallas import tpu_sc as plsc

_NC = 2
_NS = 16
_NW = _NC * _NS
_CHUNK = 64
_LANES = 16


def _sc_body(batch, num_types, dim, emb_hbm, out_hbm, buf, sem):
    c = lax.axis_index("c")
    s = lax.axis_index("s")
    wid = s * _NC + c
    rows_per_tile = batch // _NW
    pltpu.async_copy(emb_hbm, buf.at[:, 0, :], sem).wait()
    nvec = dim // _LANES
    regs = [
        buf[t, 0, pl.ds(l * _LANES, _LANES)]
        for t in range(num_types)
        for l in range(nvec)
    ]

    def _rep(r, carry):
        for t in range(num_types):
            for l in range(nvec):
                buf[t, r, pl.ds(l * _LANES, _LANES)] = regs[t * nvec + l]
        return carry

    lax.fori_loop(1, _CHUNK, _rep, 0)

    base = wid * rows_per_tile
    outs = [
        pltpu.async_copy(
            buf, out_hbm.at[:, pl.ds(base + i * _CHUNK, _CHUNK), :], sem
        )
        for i in range(rows_per_tile // _CHUNK)
    ]
    for cp in outs:
        cp.wait()


def kernel(action_mask, embedding):
    batch = action_mask.shape[0]
    num_types, dim = embedding.shape
    mesh = plsc.VectorSubcoreMesh(core_axis_name="c", subcore_axis_name="s")
    body = functools.partial(_sc_body, batch, num_types, dim)
    run = pl.kernel(
        body,
        out_type=jax.ShapeDtypeStruct((num_types, batch, dim), embedding.dtype),
        mesh=mesh,
        scratch_types=[
            pltpu.VMEM((num_types, _CHUNK, dim), embedding.dtype),
            pltpu.SemaphoreType.DMA,
        ],
        compiler_params=pltpu.CompilerParams(use_tc_tiling_on_sc=True),
    )
    return run(embedding).transpose(1, 0, 2)


# speedup vs baseline: 6.2796x; 1.7156x over previous
import jax
import jax.numpy as jnp
from jax.experimental import pallas as pl
from jax.experimental.pallas import tpu as pltpu

_BLOCK = 2048


def _bcast_body(emb_ref, out_ref, scratch_ref, sems):
    scratch_ref[...] = jnp.broadcast_to(
        emb_ref[...][:, None, :], scratch_ref.shape
    )
    batch = out_ref.shape[1]
    n_copies = batch // _BLOCK
    copies = [
        pltpu.make_async_copy(
            scratch_ref,
            out_ref.at[:, pl.ds(i * _BLOCK, _BLOCK), :],
            sems.at[i],
        )
        for i in range(n_copies)
    ]
    for c in copies:
        c.start()
    for c in copies:
        c.wait()


def kernel(action_mask, embedding):
    batch = action_mask.shape[0]
    num_types, dim = embedding.shape
    n_copies = batch // _BLOCK
    out = pl.pallas_call(
        _bcast_body,
        in_specs=[pl.BlockSpec(memory_space=pltpu.MemorySpace.VMEM)],
        out_specs=pl.BlockSpec(memory_space=pltpu.MemorySpace.HBM),
        out_shape=jax.ShapeDtypeStruct((num_types, batch, dim), embedding.dtype),
        scratch_shapes=[
            pltpu.VMEM((num_types, _BLOCK, dim), embedding.dtype),
            pltpu.SemaphoreType.DMA((n_copies,)),
        ],
    )(embedding)
    return out.transpose(1, 0, 2)
